# Initial kernel scaffold; baseline (speedup 1.0000x reference)
import jax
import jax.numpy as jnp
from jax.experimental import pallas as pl


def kernel(x, indices):
    raise NotImplementedError("write your pallas kernel here")


# speedup vs baseline: 3.8212x; 3.8212x over previous
import functools

import jax
import jax.numpy as jnp
from jax import lax
from jax.experimental import pallas as pl
from jax.experimental.pallas import tpu as pltpu
from jax.experimental.pallas import tpu_sc as plsc

B, C, H, W = 8, 96, 112, 112
KS, ST = 2, 2
HOUT, WOUT = (H - 1) * ST + KS, (W - 1) * ST + KS
ROWS = B * C
IN_ROW = H * W
OUT_ROW = HOUT * WOUT
NWORKERS = 32
ROWS_PER_W = ROWS // NWORKERS
NGROUPS = IN_ROW // 16
NZGROUPS = OUT_ROW // 16

_mesh = plsc.VectorSubcoreMesh(core_axis_name="c", subcore_axis_name="s")


@functools.partial(
    pl.kernel,
    mesh=_mesh,
    compiler_params=pltpu.CompilerParams(needs_layout_passes=False),
    out_type=jax.ShapeDtypeStruct((ROWS, OUT_ROW), jnp.float32),
    scratch_types=[
        pltpu.VMEM((IN_ROW,), jnp.float32),
        pltpu.VMEM((IN_ROW,), jnp.int32),
        pltpu.VMEM((OUT_ROW,), jnp.float32),
    ],
)
def _scatter_sorted(idx_hbm, x_hbm, out_hbm, xbuf, ibuf, obuf):
    wid = lax.axis_index("s") * 2 + lax.axis_index("c")
    lane = lax.iota(jnp.int32, 16)
    zeros16 = jnp.zeros((16,), jnp.float32)
    shift_up = jnp.minimum(lane + 1, 15)
    gather_dnums = lax.GatherDimensionNumbers(
        offset_dims=(), collapsed_slice_dims=(0,), start_index_map=(0,))
    lane15 = lane == 15

    def shift_lanes_up(v):
        return lax.gather(v, shift_up[:, None], gather_dnums, (1,),
                          mode=lax.GatherScatterMode.PROMISE_IN_BOUNDS)

    def do_row(r, _):
        row = wid * ROWS_PER_W + r
        pltpu.sync_copy(x_hbm.at[row], xbuf)
        pltpu.sync_copy(idx_hbm.at[row], ibuf)

        def zero_grp(z, _):
            obuf[pl.ds(z * 16, 16)] = zeros16
            return _

        lax.fori_loop(0, NZGROUPS, zero_grp, 0, unroll=8)

        def scatter_grp(g, _):
            idxv = ibuf[pl.ds(g * 16, 16)]
            xv = xbuf[pl.ds(g * 16, 16)]
            nxt = shift_lanes_up(idxv)
            keep = jnp.logical_or(idxv != nxt, lane15)
            plsc.store_scatter(obuf, [idxv], xv, mask=keep)
            return _

        lax.fori_loop(0, NGROUPS, scatter_grp, 0)
        pltpu.sync_copy(obuf, out_hbm.at[row])
        return _

    lax.fori_loop(0, ROWS_PER_W, do_row, 0)


def kernel(x, indices):
    xf = x.reshape(ROWS * IN_ROW)
    idxf = indices.astype(jnp.int32).reshape(ROWS, IN_ROW)
    row = jax.lax.broadcasted_iota(jnp.int32, (ROWS, IN_ROW), 0)
    idxw = jnp.where(idxf < 0, idxf + OUT_ROW, idxf)
    valid = (idxw >= 0) & (idxw < OUT_ROW) & (row >= 0) & (row < ROWS)
    key = jnp.where(valid, row * OUT_ROW + idxw, -1).reshape(ROWS * IN_ROW)
    skey, sval = lax.sort((key, xf), dimension=0, is_stable=False,
                          num_keys=1)
    sidx = (skey.reshape(ROWS, IN_ROW)
            - jnp.arange(ROWS, dtype=jnp.int32)[:, None] * OUT_ROW)
    out = _scatter_sorted(sidx, sval.reshape(ROWS, IN_ROW))
    return out.reshape(B, C, HOUT, WOUT)


# speedup vs baseline: 3.8323x; 1.0029x over previous
import functools

import jax
import jax.numpy as jnp
from jax import lax
from jax.experimental import pallas as pl
from jax.experimental.pallas import tpu as pltpu
from jax.experimental.pallas import tpu_sc as plsc

B, C, H, W = 8, 96, 112, 112
KS, ST = 2, 2
HOUT, WOUT = (H - 1) * ST + KS, (W - 1) * ST + KS
ROWS = B * C
IN_ROW = H * W
OUT_ROW = HOUT * WOUT
NWORKERS = 32
ROWS_PER_W = ROWS // NWORKERS
NPAIRS = ROWS_PER_W // 2
NGROUPS = IN_ROW // 16
NZGROUPS = OUT_ROW // 16

_mesh = plsc.VectorSubcoreMesh(core_axis_name="c", subcore_axis_name="s")


@functools.partial(
    pl.kernel,
    mesh=_mesh,
    compiler_params=pltpu.CompilerParams(needs_layout_passes=False),
    out_type=jax.ShapeDtypeStruct((ROWS, OUT_ROW), jnp.float32),
    scratch_types=[
        pltpu.VMEM((IN_ROW,), jnp.float32),
        pltpu.VMEM((IN_ROW,), jnp.int32),
        pltpu.VMEM((OUT_ROW,), jnp.float32),
        pltpu.VMEM((OUT_ROW,), jnp.float32),
        pltpu.SemaphoreType.DMA,
        pltpu.SemaphoreType.DMA,
    ],
)
def _scatter_sorted(idx_hbm, x_hbm, out_hbm, xbuf, ibuf, obuf0, obuf1,
                    sem0, sem1):
    wid = lax.axis_index("s") * 2 + lax.axis_index("c")
    base = wid * ROWS_PER_W
    lane = lax.iota(jnp.int32, 16)
    zeros16 = jnp.zeros((16,), jnp.float32)
    shift_up = jnp.minimum(lane + 1, 15)
    gather_dnums = lax.GatherDimensionNumbers(
        offset_dims=(), collapsed_slice_dims=(0,), start_index_map=(0,))
    lane15 = lane == 15

    def shift_lanes_up(v):
        return lax.gather(v, shift_up[:, None], gather_dnums, (1,),
                          mode=lax.GatherScatterMode.PROMISE_IN_BOUNDS)

    def do_row(row, obuf, sem, first):
        pltpu.sync_copy(x_hbm.at[row], xbuf)
        pltpu.sync_copy(idx_hbm.at[row], ibuf)

        @pl.when(jnp.logical_not(first))
        def _wait_prev():
            pltpu.make_async_copy(obuf, out_hbm.at[row], sem).wait()

        def zero_grp(z, _):
            obuf[pl.ds(z * 16, 16)] = zeros16
            return _

        lax.fori_loop(0, NZGROUPS, zero_grp, 0, unroll=8)

        def scatter_grp(g, _):
            idxv = ibuf[pl.ds(g * 16, 16)]
            xv = xbuf[pl.ds(g * 16, 16)]
            nxt = shift_lanes_up(idxv)
            keep = jnp.logical_or(idxv != nxt, lane15)
            plsc.store_scatter(obuf, [idxv], xv, mask=keep)
            return _

        lax.fori_loop(0, NGROUPS, scatter_grp, 0)
        pltpu.async_copy(obuf, out_hbm.at[row], sem)

    def do_pair(k, _):
        do_row(base + 2 * k, obuf0, sem0, k == 0)
        do_row(base + 2 * k + 1, obuf1, sem1, k == 0)
        return _

    lax.fori_loop(0, NPAIRS, do_pair, 0)
    pltpu.make_async_copy(obuf0, out_hbm.at[base], sem0).wait()
    pltpu.make_async_copy(obuf1, out_hbm.at[base], sem1).wait()


def kernel(x, indices):
    xf = x.reshape(ROWS * IN_ROW)
    idxf = indices.astype(jnp.int32).reshape(ROWS, IN_ROW)
    row = jax.lax.broadcasted_iota(jnp.int32, (ROWS, IN_ROW), 0)
    idxw = jnp.where(idxf < 0, idxf + OUT_ROW, idxf)
    valid = (idxw >= 0) & (idxw < OUT_ROW) & (row >= 0) & (row < ROWS)
    key = jnp.where(valid, row * OUT_ROW + idxw, -1).reshape(ROWS * IN_ROW)
    skey, sval = lax.sort((key, xf), dimension=0, is_stable=False,
                          num_keys=1)
    sidx = (skey.reshape(ROWS, IN_ROW)
            - jnp.arange(ROWS, dtype=jnp.int32)[:, None] * OUT_ROW)
    out = _scatter_sorted(sidx, sval.reshape(ROWS, IN_ROW))
    return out.reshape(B, C, HOUT, WOUT)
